# baseline (speedup 1.0000x reference)
import functools

import jax
import jax.numpy as jnp
from jax import lax
from jax.experimental import pallas as pl
from jax.experimental.pallas import tpu as pltpu
from jax.experimental.pallas import tpu_sc as plsc

NUM_USERS = 8192
NUM_ITEMS = 8192
N = NUM_USERS + NUM_ITEMS
NNZ = 268435
D = 64
BATCH = 4096
N_LAYERS = 3

NC = 2
NS = 16
L = 16

CH = 128
E_PER_SUB = -(-NNZ // NS)
NCHUNK = -(-E_PER_SUB // CH)
E_SUB_PAD = NCHUNK * CH
E_PAD = E_SUB_PAD * NS

ROWS_PER_SUB = N // NS
BPW = BATCH // (NC * NS)

NBUF = 4
NGRP = NCHUNK // NBUF
EB = 3
assert NCHUNK % NBUF == 0

_mesh = plsc.VectorSubcoreMesh(core_axis_name="c", subcore_axis_name="s")
_params = pltpu.CompilerParams(use_tc_tiling_on_sc=False,
                               needs_layout_passes=False)


@functools.partial(
    pl.kernel,
    out_type=jax.ShapeDtypeStruct((NC, N_LAYERS + 1, N, D), jnp.float32),
    mesh=_mesh,
    scratch_types=[
        pltpu.VMEM((EB, NBUF, CH), jnp.int32),
        pltpu.VMEM((EB, NBUF, CH), jnp.int32),
        pltpu.VMEM((EB, NBUF, CH), jnp.float32),
        pltpu.VMEM((NBUF, CH, D), jnp.float32),
        pltpu.VMEM((2, CH, D), jnp.float32),
        pltpu.VMEM((CH, D), jnp.float32),
        pltpu.VMEM_SHARED((N, D), jnp.float32),
        pltpu.SemaphoreType.DMA((EB,)),
        pltpu.SemaphoreType.DMA((NBUF,)),
        pltpu.SemaphoreType.DMA((2,)),
        pltpu.SemaphoreType.DMA,
        pltpu.SemaphoreType.DMA,
    ],
    compiler_params=_params,
)
def _propagate(x0s_hbm, col_hbm, row_hbm, val_hbm, xs_hbm,
               colb, rowb, valb, gbuf, sbuf, zbuf, acc_sh,
               esem, gsem, ssem, wsem, wbsem):
    c = lax.axis_index("c")
    s = lax.axis_index("s")

    def _eb_issue(g, u):
        pltpu.async_copy(col_hbm.at[c, s, pl.ds(g * NBUF, NBUF)],
                         colb.at[u], esem.at[u])
        pltpu.async_copy(row_hbm.at[c, s, pl.ds(g * NBUF, NBUF)],
                         rowb.at[u], esem.at[u])
        pltpu.async_copy(val_hbm.at[c, s, pl.ds(g * NBUF, NBUF)],
                         valb.at[u], esem.at[u])

    def _eb_wait(g, u):
        pltpu.make_async_copy(col_hbm.at[c, s, pl.ds(g * NBUF, NBUF)],
                              colb.at[u], esem.at[u]).wait()
        pltpu.make_async_copy(row_hbm.at[c, s, pl.ds(g * NBUF, NBUF)],
                              rowb.at[u], esem.at[u]).wait()
        pltpu.make_async_copy(val_hbm.at[c, s, pl.ds(g * NBUF, NBUF)],
                              valb.at[u], esem.at[u]).wait()

    def _zero_acc():
        for i in range(ROWS_PER_SUB // CH):
            pltpu.async_copy(
                zbuf, acc_sh.at[pl.ds(s * ROWS_PER_SUB + i * CH, CH)], wsem)
        for i in range(ROWS_PER_SUB // CH):
            pltpu.make_async_copy(
                zbuf, acc_sh.at[pl.ds(s * ROWS_PER_SUB + i * CH, CH)],
                wsem).wait()

    def _zrow(e, carry):
        for q in range(D // L):
            zbuf[e, pl.ds(q * L, L)] = jnp.zeros((L,), jnp.float32)
        return carry
    lax.fori_loop(0, CH, _zrow, 0)
    own = pl.ds(s * ROWS_PER_SUB, ROWS_PER_SUB)
    pltpu.async_copy(x0s_hbm.at[c, own], xs_hbm.at[c, 0, own], wbsem)
    _zero_acc()
    pltpu.make_async_copy(x0s_hbm.at[c, own], xs_hbm.at[c, 0, own],
                          wbsem).wait()
    plsc.subcore_barrier()

    def _layer(l, carry):
        src = xs_hbm.at[c, l]

        _eb_issue(0, 0)
        _eb_issue(1, 1)
        _eb_wait(0, 0)
        _eb_wait(1, 1)
        for b in range(NBUF):
            pltpu.async_copy(src.at[colb.at[0, b]], gbuf.at[b], gsem.at[b])

        def _group(g, carry):
            u = lax.rem(g, EB)
            un = lax.rem(g + 1, EB)
            uf = lax.rem(g + 2, EB)
            for b in range(NBUF):
                sb = b % 2
                pltpu.make_async_copy(src.at[colb.at[u, b]], gbuf.at[b],
                                      gsem.at[b]).wait()
                if b >= 2:
                    pltpu.make_async_copy(
                        sbuf.at[sb], acc_sh.at[rowb.at[u, b - 2]],
                        ssem.at[sb]).wait()
                else:
                    @pl.when(g > 0)
                    def _():
                        pltpu.make_async_copy(
                            sbuf.at[sb], acc_sh.at[rowb.at[uf, b + 2]],
                            ssem.at[sb]).wait()

                @plsc.parallel_loop(0, CH, step=L)
                def _scale(e0):
                    vv = valb[u, b, pl.ds(e0, L)]
                    for t in range(L):
                        v = vv[t]
                        for q in range(D // L):
                            sbuf[sb, e0 + t, pl.ds(q * L, L)] = \
                                gbuf[b, e0 + t, pl.ds(q * L, L)] * v
                pltpu.async_copy(sbuf.at[sb], acc_sh.at[rowb.at[u, b]],
                                 ssem.at[sb], add=True)
                @pl.when(g < NGRP - 1)
                def _():
                    pltpu.async_copy(src.at[colb.at[un, b]], gbuf.at[b],
                                     gsem.at[b])
                if b == 1:
                    @pl.when(g < NGRP - 2)
                    def _():
                        _eb_issue(g + 2, uf)

            @pl.when(g < NGRP - 2)
            def _():
                _eb_wait(g + 2, uf)
            return carry
        lax.fori_loop(0, NGRP, _group, 0)

        ul = (NGRP - 1) % EB
        pltpu.make_async_copy(sbuf.at[0], acc_sh.at[rowb.at[ul, 2]],
                              ssem.at[0]).wait()
        pltpu.make_async_copy(sbuf.at[1], acc_sh.at[rowb.at[ul, 3]],
                              ssem.at[1]).wait()
        plsc.subcore_barrier()

        pltpu.async_copy(acc_sh.at[own], xs_hbm.at[c, l + 1, own], wbsem)
        pltpu.make_async_copy(acc_sh.at[own], xs_hbm.at[c, l + 1, own],
                              wbsem).wait()
        _zero_acc()
        plsc.subcore_barrier()
        return carry
    lax.fori_loop(0, N_LAYERS, _layer, 0)


@functools.partial(
    pl.kernel,
    out_type=jax.ShapeDtypeStruct((BATCH,), jnp.float32),
    mesh=_mesh,
    scratch_types=[
        pltpu.VMEM((BPW,), jnp.int32),
        pltpu.VMEM((BPW,), jnp.int32),
        pltpu.VMEM((BPW,), jnp.int32),
        pltpu.VMEM((NUM_ITEMS,), jnp.int32),
        pltpu.VMEM((BPW, D), jnp.float32),
        pltpu.VMEM((BPW, D), jnp.float32),
        pltpu.VMEM((4, BPW, D), jnp.float32),
        pltpu.VMEM((BPW,), jnp.float32),
        pltpu.SemaphoreType.DMA((4,)),
    ],
    compiler_params=_params,
)
def _readout(users_hbm, items_hbm, map_hbm, xs_hbm, gamma_hbm,
             uidx, iidx, midx, map_v, uacc, iacc, tmp8, gout, sem):
    c = lax.axis_index("c")
    s = lax.axis_index("s")
    wid = s * NC + c
    base = wid * BPW
    pltpu.sync_copy(users_hbm.at[pl.ds(base, BPW)], uidx)
    pltpu.sync_copy(items_hbm.at[pl.ds(base, BPW)], iidx)
    pltpu.sync_copy(map_hbm, map_v)

    for g in range(BPW // L):
        ivec = iidx[pl.ds(g * L, L)]
        m = plsc.load_gather(map_v, [ivec])
        midx[pl.ds(g * L, L)] = m + NUM_USERS
        iidx[pl.ds(g * L, L)] = ivec + NUM_USERS

    tables = [xs_hbm.at[cc, ll]
              for ll in range(N_LAYERS + 1) for cc in range(NC)]

    def _acc_into(accbuf, gathered):
        @plsc.parallel_loop(0, BPW, step=L)
        def _zrow(e0):
            for t in range(L):
                for q in range(D // L):
                    accbuf[e0 + t, pl.ds(q * L, L)] = jnp.zeros(
                        (L,), jnp.float32)

        for r in range(2):
            part = gathered[r * 4:(r + 1) * 4]
            for k, tbl in enumerate(part):
                pltpu.async_copy(tbl, tmp8.at[k], sem.at[k])
            for k, tbl in enumerate(part):
                pltpu.make_async_copy(tbl, tmp8.at[k], sem.at[k]).wait()

            def _slab(k, carry):
                @plsc.parallel_loop(0, BPW, step=L)
                def _addrow(e0):
                    for t in range(L):
                        for q in range(D // L):
                            sl = pl.ds(q * L, L)
                            accbuf[e0 + t, sl] = \
                                accbuf[e0 + t, sl] + tmp8[k, e0 + t, sl]
                return carry
            lax.fori_loop(0, 4, _slab, 0)

    _acc_into(uacc, [tb.at[uidx] for tb in tables])
    _acc_into(iacc, [tb.at[iidx if k % 2 == 0 else midx]
                     for k, tb in enumerate(tables)])

    lane = lax.broadcasted_iota(jnp.int32, (L,), 0)

    def _dot(g, carry):
        gvec = jnp.zeros((L,), jnp.float32)
        for t in range(L):
            e = g * L + t
            p = uacc[e, pl.ds(0, L)] * iacc[e, pl.ds(0, L)]
            for q in range(1, D // L):
                sl = pl.ds(q * L, L)
                p = p + uacc[e, sl] * iacc[e, sl]
            gvec = jnp.where(lane == t, jnp.sum(p) * (1.0 / 64.0), gvec)
        gout[pl.ds(g * L, L)] = gvec
        return carry
    lax.fori_loop(0, BPW // L, _dot, 0)
    pltpu.sync_copy(gout, gamma_hbm.at[pl.ds(base, BPW)])


def _pad_edges(a):
    return jnp.pad(a, (0, E_PAD - NNZ)).reshape(NS, NCHUNK, CH)


def kernel(users, items, user_emb_0, user_emb_1, item_emb_0, item_emb_1,
           edge_row_0, edge_col_0, edge_val_0,
           edge_row_1, edge_col_1, edge_val_1, map_list):
    x0s = jnp.stack([jnp.concatenate([user_emb_0, item_emb_0], axis=0),
                     jnp.concatenate([user_emb_1, item_emb_1], axis=0)])
    colp = jnp.stack([_pad_edges(edge_col_0), _pad_edges(edge_col_1)])
    rowp = jnp.stack([_pad_edges(edge_row_0), _pad_edges(edge_row_1)])
    valp = jnp.stack([_pad_edges(edge_val_0), _pad_edges(edge_val_1)])
    xs = _propagate(x0s, colp, rowp, valp)
    return _readout(users, items, map_list, xs)


# speedup vs baseline: 2.0031x; 2.0031x over previous
import functools

import jax
import jax.numpy as jnp
from jax import lax
from jax.experimental import pallas as pl
from jax.experimental.pallas import tpu as pltpu
from jax.experimental.pallas import tpu_sc as plsc

NUM_USERS = 8192
NUM_ITEMS = 8192
N = NUM_USERS + NUM_ITEMS
NNZ = 268435
D = 64
BATCH = 4096
N_LAYERS = 3

NC = 2
NS = 16
L = 16

CH = 128
E_PER_SUB = -(-NNZ // NS)
NCHUNK = -(-E_PER_SUB // CH)
E_SUB_PAD = NCHUNK * CH
E_PAD = E_SUB_PAD * NS

ROWS_PER_SUB = N // NS
BPW = BATCH // (NC * NS)

NBUF = 4
NGRP = NCHUNK // NBUF
EB = 3
assert NCHUNK % NBUF == 0

_mesh = plsc.VectorSubcoreMesh(core_axis_name="c", subcore_axis_name="s")
_params = pltpu.CompilerParams(use_tc_tiling_on_sc=False,
                               needs_layout_passes=False)


@functools.partial(
    pl.kernel,
    out_type=jax.ShapeDtypeStruct((NC, N_LAYERS, N, D), jnp.bfloat16),
    mesh=_mesh,
    scratch_types=[
        pltpu.VMEM((EB, NBUF, CH), jnp.int32),
        pltpu.VMEM((EB, NBUF, CH), jnp.int32),
        pltpu.VMEM((EB, NBUF, CH), jnp.float32),
        pltpu.VMEM((NBUF, CH, D), jnp.bfloat16),
        pltpu.VMEM((2, CH, D), jnp.float32),
        pltpu.VMEM((CH, D), jnp.float32),
        pltpu.VMEM((2, CH, D), jnp.bfloat16),
        pltpu.VMEM_SHARED((N, D), jnp.float32),
        pltpu.SemaphoreType.DMA((EB,)),
        pltpu.SemaphoreType.DMA((NBUF,)),
        pltpu.SemaphoreType.DMA((2,)),
        pltpu.SemaphoreType.DMA,
        pltpu.SemaphoreType.DMA((2,)),
        pltpu.SemaphoreType.DMA((2,)),
    ],
    compiler_params=_params,
)
def _propagate(x0s_hbm, col_hbm, row_hbm, val_hbm, xs_hbm,
               colb, rowb, valb, gbuf, sbuf, zbuf, wbuf, acc_sh,
               esem, gsem, ssem, wsem, wisem, wosem):
    c = lax.axis_index("c")
    s = lax.axis_index("s")

    def _eb_issue(g, u):
        pltpu.async_copy(col_hbm.at[c, s, pl.ds(g * NBUF, NBUF)],
                         colb.at[u], esem.at[u])
        pltpu.async_copy(row_hbm.at[c, s, pl.ds(g * NBUF, NBUF)],
                         rowb.at[u], esem.at[u])
        pltpu.async_copy(val_hbm.at[c, s, pl.ds(g * NBUF, NBUF)],
                         valb.at[u], esem.at[u])

    def _eb_wait(g, u):
        pltpu.make_async_copy(col_hbm.at[c, s, pl.ds(g * NBUF, NBUF)],
                              colb.at[u], esem.at[u]).wait()
        pltpu.make_async_copy(row_hbm.at[c, s, pl.ds(g * NBUF, NBUF)],
                              rowb.at[u], esem.at[u]).wait()
        pltpu.make_async_copy(val_hbm.at[c, s, pl.ds(g * NBUF, NBUF)],
                              valb.at[u], esem.at[u]).wait()

    def _zero_acc():
        for i in range(ROWS_PER_SUB // CH):
            pltpu.async_copy(
                zbuf, acc_sh.at[pl.ds(s * ROWS_PER_SUB + i * CH, CH)], wsem)
        for i in range(ROWS_PER_SUB // CH):
            pltpu.make_async_copy(
                zbuf, acc_sh.at[pl.ds(s * ROWS_PER_SUB + i * CH, CH)],
                wsem).wait()

    def _zrow(e, carry):
        for q in range(D // L):
            zbuf[e, pl.ds(q * L, L)] = jnp.zeros((L,), jnp.float32)
        return carry
    lax.fori_loop(0, CH, _zrow, 0)
    _zero_acc()
    plsc.subcore_barrier()

    for l in range(N_LAYERS):
        src = x0s_hbm.at[c] if l == 0 else xs_hbm.at[c, l - 1]

        _eb_issue(0, 0)
        _eb_issue(1, 1)
        _eb_wait(0, 0)
        _eb_wait(1, 1)
        for b in range(NBUF):
            pltpu.async_copy(src.at[colb.at[0, b]], gbuf.at[b], gsem.at[b])

        def _chunk(j, carry):
            g = lax.shift_right_logical(j, 2)
            b = lax.bitwise_and(j, NBUF - 1)
            sb = lax.bitwise_and(j, 1)
            u = lax.rem(g, EB)
            un = lax.rem(g + 1, EB)
            uf = lax.rem(g + 2, EB)
            pltpu.make_async_copy(src.at[colb.at[u, b]], gbuf.at[b],
                                  gsem.at[b]).wait()

            @pl.when(j >= 2)
            def _():
                b2 = lax.bitwise_and(j - 2, NBUF - 1)
                u2 = lax.rem(lax.shift_right_logical(j - 2, 2), EB)
                pltpu.make_async_copy(
                    sbuf.at[sb], acc_sh.at[rowb.at[u2, b2]],
                    ssem.at[sb]).wait()

            @plsc.parallel_loop(0, CH, step=L, unroll=2)
            def _scale(e0):
                vv = valb[u, b, pl.ds(e0, L)]
                for t in range(L):
                    v = vv[t]
                    for h in range(D // (2 * L)):
                        w = gbuf[b, e0 + t, pl.ds(h * 2 * L, 2 * L)]
                        pa, pb = plsc.unpack(
                            w, format=plsc.PackFormat.INTERLEAVED)
                        sbuf[sb, e0 + t, pl.ds(h * 2 * L, L)] = pa * v
                        sbuf[sb, e0 + t, pl.ds(h * 2 * L + L, L)] = pb * v
            pltpu.async_copy(sbuf.at[sb], acc_sh.at[rowb.at[u, b]],
                             ssem.at[sb], add=True)

            @pl.when(j + NBUF < NCHUNK)
            def _():
                pltpu.async_copy(src.at[colb.at[un, b]], gbuf.at[b],
                                 gsem.at[b])

            @pl.when(jnp.logical_and(b == 1, g < NGRP - 2))
            def _():
                _eb_issue(g + 2, uf)

            @pl.when(jnp.logical_and(b == NBUF - 1, g < NGRP - 2))
            def _():
                _eb_wait(g + 2, uf)
            return carry
        lax.fori_loop(0, NCHUNK, _chunk, 0)

        ul = (NGRP - 1) % EB
        pltpu.make_async_copy(sbuf.at[0], acc_sh.at[rowb.at[ul, 2]],
                              ssem.at[0]).wait()
        pltpu.make_async_copy(sbuf.at[1], acc_sh.at[rowb.at[ul, 3]],
                              ssem.at[1]).wait()
        plsc.subcore_barrier()

        nw = ROWS_PER_SUB // CH

        def _wsl(i):
            return pl.ds(s * ROWS_PER_SUB + i * CH, CH)

        def _wb_step(i, p):
            pltpu.make_async_copy(acc_sh.at[_wsl(i)], sbuf.at[p],
                                  wisem.at[p]).wait()

            @pl.when(i + 1 < nw)
            def _():
                pltpu.async_copy(acc_sh.at[_wsl(i + 1)], sbuf.at[1 - p],
                                 wisem.at[1 - p])
            if l < N_LAYERS - 1:
                pltpu.async_copy(zbuf, acc_sh.at[_wsl(i)], wsem)

            @pl.when(i >= 2)
            def _():
                pltpu.make_async_copy(
                    wbuf.at[p], xs_hbm.at[c, l, _wsl(i - 2)],
                    wosem.at[p]).wait()

            @plsc.parallel_loop(0, CH, step=L)
            def _packrow(e0):
                for t in range(L):
                    for h in range(D // (2 * L)):
                        pa = sbuf[p, e0 + t, pl.ds(h * 2 * L, L)]
                        pb = sbuf[p, e0 + t, pl.ds(h * 2 * L + L, L)]
                        wbuf[p, e0 + t, pl.ds(h * 2 * L, 2 * L)] = plsc.pack(
                            pa, pb, format=plsc.PackFormat.INTERLEAVED)
            pltpu.async_copy(wbuf.at[p], xs_hbm.at[c, l, _wsl(i)],
                             wosem.at[p])

        pltpu.async_copy(acc_sh.at[_wsl(0)], sbuf.at[0], wisem.at[0])

        def _wb2(i2, carry2):
            _wb_step(i2 * 2, 0)
            _wb_step(i2 * 2 + 1, 1)
            return carry2
        lax.fori_loop(0, nw // 2, _wb2, 0)
        for i in (nw - 2, nw - 1):
            p = i % 2
            pltpu.make_async_copy(wbuf.at[p], xs_hbm.at[c, l, _wsl(i)],
                                  wosem.at[p]).wait()
        if l < N_LAYERS - 1:
            for i in range(nw):
                pltpu.make_async_copy(zbuf, acc_sh.at[_wsl(i)], wsem).wait()
        plsc.subcore_barrier()


@functools.partial(
    pl.kernel,
    out_type=jax.ShapeDtypeStruct((BATCH,), jnp.float32),
    mesh=_mesh,
    scratch_types=[
        pltpu.VMEM((BPW,), jnp.int32),
        pltpu.VMEM((BPW,), jnp.int32),
        pltpu.VMEM((BPW,), jnp.int32),
        pltpu.VMEM((NUM_ITEMS,), jnp.int32),
        pltpu.VMEM((BPW, D), jnp.float32),
        pltpu.VMEM((BPW, D), jnp.float32),
        pltpu.VMEM((8, BPW, D), jnp.bfloat16),
        pltpu.VMEM((BPW,), jnp.float32),
        pltpu.SemaphoreType.DMA((8,)),
    ],
    compiler_params=_params,
)
def _readout(users_hbm, items_hbm, map_hbm, x0s_hbm, xs_hbm, gamma_hbm,
             uidx, iidx, midx, map_v, uacc, iacc, tmp8, gout, sem):
    c = lax.axis_index("c")
    s = lax.axis_index("s")
    wid = s * NC + c
    base = wid * BPW
    pltpu.sync_copy(users_hbm.at[pl.ds(base, BPW)], uidx)
    pltpu.sync_copy(items_hbm.at[pl.ds(base, BPW)], iidx)
    pltpu.sync_copy(map_hbm, map_v)

    for g in range(BPW // L):
        ivec = iidx[pl.ds(g * L, L)]
        m = plsc.load_gather(map_v, [ivec])
        midx[pl.ds(g * L, L)] = m + NUM_USERS
        iidx[pl.ds(g * L, L)] = ivec + NUM_USERS

    tables = [x0s_hbm.at[0], x0s_hbm.at[1]] + \
        [xs_hbm.at[cc, ll] for ll in range(N_LAYERS) for cc in range(NC)]

    def _acc_into(accbuf, gathered):
        for k, tbl in enumerate(gathered):
            pltpu.async_copy(tbl, tmp8.at[k], sem.at[k])
        for k, tbl in enumerate(gathered):
            pltpu.make_async_copy(tbl, tmp8.at[k], sem.at[k]).wait()

        @plsc.parallel_loop(0, BPW, step=L)
        def _zrow(e0):
            for t in range(L):
                for q in range(D // L):
                    accbuf[e0 + t, pl.ds(q * L, L)] = jnp.zeros(
                        (L,), jnp.float32)

        def _slab(k, carry):
            @plsc.parallel_loop(0, BPW, step=L)
            def _addrow(e0):
                for t in range(L):
                    for h in range(D // (2 * L)):
                        w = tmp8[k, e0 + t, pl.ds(h * 2 * L, 2 * L)]
                        pa, pb = plsc.unpack(
                            w, format=plsc.PackFormat.INTERLEAVED)
                        sla = pl.ds(h * 2 * L, L)
                        slb = pl.ds(h * 2 * L + L, L)
                        accbuf[e0 + t, sla] = accbuf[e0 + t, sla] + pa
                        accbuf[e0 + t, slb] = accbuf[e0 + t, slb] + pb
            return carry
        lax.fori_loop(0, 8, _slab, 0)

    _acc_into(uacc, [tb.at[uidx] for tb in tables])
    _acc_into(iacc, [tb.at[iidx if k % 2 == 0 else midx]
                     for k, tb in enumerate(tables)])

    lane = lax.broadcasted_iota(jnp.int32, (L,), 0)

    def _dot(g, carry):
        gvec = jnp.zeros((L,), jnp.float32)
        for t in range(L):
            e = g * L + t
            p = uacc[e, pl.ds(0, L)] * iacc[e, pl.ds(0, L)]
            for q in range(1, D // L):
                sl = pl.ds(q * L, L)
                p = p + uacc[e, sl] * iacc[e, sl]
            gvec = jnp.where(lane == t, jnp.sum(p) * (1.0 / 64.0), gvec)
        gout[pl.ds(g * L, L)] = gvec
        return carry
    lax.fori_loop(0, BPW // L, _dot, 0)
    pltpu.sync_copy(gout, gamma_hbm.at[pl.ds(base, BPW)])


def _pad_edges(a):
    return jnp.pad(a, (0, E_PAD - NNZ)).reshape(NS, NCHUNK, CH)


def kernel(users, items, user_emb_0, user_emb_1, item_emb_0, item_emb_1,
           edge_row_0, edge_col_0, edge_val_0,
           edge_row_1, edge_col_1, edge_val_1, map_list):
    x0s = jnp.stack([jnp.concatenate([user_emb_0, item_emb_0], axis=0),
                     jnp.concatenate([user_emb_1, item_emb_1], axis=0)])
    x0b = (x0s.reshape(NC, N, 2, 2, L).transpose(0, 1, 2, 4, 3)
           .reshape(NC, N, D).astype(jnp.bfloat16))
    colp = jnp.stack([_pad_edges(edge_col_0), _pad_edges(edge_col_1)])
    rowp = jnp.stack([_pad_edges(edge_row_0), _pad_edges(edge_row_1)])
    valp = jnp.stack([_pad_edges(edge_val_0), _pad_edges(edge_val_1)])
    xs = _propagate(x0b, colp, rowp, valp)
    return _readout(users, items, map_list, x0b, xs)


# speedup vs baseline: 2.0149x; 1.0059x over previous
import functools

import jax
import jax.numpy as jnp
from jax import lax
from jax.experimental import pallas as pl
from jax.experimental.pallas import tpu as pltpu
from jax.experimental.pallas import tpu_sc as plsc

NUM_USERS = 8192
NUM_ITEMS = 8192
N = NUM_USERS + NUM_ITEMS
NNZ = 268435
D = 64
BATCH = 4096
N_LAYERS = 3

NC = 2
NS = 16
L = 16

CH = 128
E_PER_SUB = -(-NNZ // NS)
NCHUNK = -(-E_PER_SUB // CH)
E_SUB_PAD = NCHUNK * CH
E_PAD = E_SUB_PAD * NS

ROWS_PER_SUB = N // NS
BPW = BATCH // (NC * NS)

NBUF = 4
NGRP = NCHUNK // NBUF
EB = 3
assert NCHUNK % NBUF == 0

_mesh = plsc.VectorSubcoreMesh(core_axis_name="c", subcore_axis_name="s")
_params = pltpu.CompilerParams(use_tc_tiling_on_sc=False,
                               needs_layout_passes=False)


@functools.partial(
    pl.kernel,
    out_type=jax.ShapeDtypeStruct((NC, N_LAYERS, N, D), jnp.bfloat16),
    mesh=_mesh,
    scratch_types=[
        pltpu.VMEM((EB, NBUF, CH), jnp.int32),
        pltpu.VMEM((EB, NBUF, CH), jnp.int32),
        pltpu.VMEM((EB, NBUF, CH), jnp.float32),
        pltpu.VMEM((NBUF, CH, D), jnp.bfloat16),
        pltpu.VMEM((2, CH, D), jnp.float32),
        pltpu.VMEM((CH, D), jnp.float32),
        pltpu.VMEM((2, CH, D), jnp.bfloat16),
        pltpu.VMEM_SHARED((N, D), jnp.float32),
        pltpu.SemaphoreType.DMA((EB,)),
        pltpu.SemaphoreType.DMA((NBUF,)),
        pltpu.SemaphoreType.DMA((2,)),
        pltpu.SemaphoreType.DMA,
        pltpu.SemaphoreType.DMA((2,)),
        pltpu.SemaphoreType.DMA((2,)),
    ],
    compiler_params=_params,
)
def _propagate(x0s_hbm, col_hbm, row_hbm, val_hbm, xs_hbm,
               colb, rowb, valb, gbuf, sbuf, zbuf, wbuf, acc_sh,
               esem, gsem, ssem, wsem, wisem, wosem):
    c = lax.axis_index("c")
    s = lax.axis_index("s")

    def _eb_issue(g, u):
        pltpu.async_copy(col_hbm.at[c, s, pl.ds(g * NBUF, NBUF)],
                         colb.at[u], esem.at[u])
        pltpu.async_copy(row_hbm.at[c, s, pl.ds(g * NBUF, NBUF)],
                         rowb.at[u], esem.at[u])
        pltpu.async_copy(val_hbm.at[c, s, pl.ds(g * NBUF, NBUF)],
                         valb.at[u], esem.at[u])

    def _eb_wait(g, u):
        pltpu.make_async_copy(col_hbm.at[c, s, pl.ds(g * NBUF, NBUF)],
                              colb.at[u], esem.at[u]).wait()
        pltpu.make_async_copy(row_hbm.at[c, s, pl.ds(g * NBUF, NBUF)],
                              rowb.at[u], esem.at[u]).wait()
        pltpu.make_async_copy(val_hbm.at[c, s, pl.ds(g * NBUF, NBUF)],
                              valb.at[u], esem.at[u]).wait()

    def _zero_acc():
        for i in range(ROWS_PER_SUB // CH):
            pltpu.async_copy(
                zbuf, acc_sh.at[pl.ds(s * ROWS_PER_SUB + i * CH, CH)], wsem)
        for i in range(ROWS_PER_SUB // CH):
            pltpu.make_async_copy(
                zbuf, acc_sh.at[pl.ds(s * ROWS_PER_SUB + i * CH, CH)],
                wsem).wait()

    def _zrow(e, carry):
        for q in range(D // L):
            zbuf[e, pl.ds(q * L, L)] = jnp.zeros((L,), jnp.float32)
        return carry
    lax.fori_loop(0, CH, _zrow, 0)
    _zero_acc()
    plsc.subcore_barrier()

    for l in range(N_LAYERS):
        src = x0s_hbm.at[c] if l == 0 else xs_hbm.at[c, l - 1]

        _eb_issue(0, 0)
        _eb_issue(1, 1)
        _eb_wait(0, 0)
        _eb_wait(1, 1)
        for b in range(NBUF):
            pltpu.async_copy(src.at[colb.at[0, b]], gbuf.at[b], gsem.at[b])

        def _chunk(j, carry):
            g = lax.shift_right_logical(j, 2)
            b = lax.bitwise_and(j, NBUF - 1)
            sb = lax.bitwise_and(j, 1)
            u = lax.rem(g, EB)
            un = lax.rem(g + 1, EB)
            uf = lax.rem(g + 2, EB)
            pltpu.make_async_copy(src.at[colb.at[u, b]], gbuf.at[b],
                                  gsem.at[b]).wait()

            @pl.when(j >= 2)
            def _():
                b2 = lax.bitwise_and(j - 2, NBUF - 1)
                u2 = lax.rem(lax.shift_right_logical(j - 2, 2), EB)
                pltpu.make_async_copy(
                    sbuf.at[sb], acc_sh.at[rowb.at[u2, b2]],
                    ssem.at[sb]).wait()

            @plsc.parallel_loop(0, CH, step=L)
            def _scale(e0):
                vv = valb[u, b, pl.ds(e0, L)]
                for t in range(L):
                    v = vv[t]
                    for h in range(D // (2 * L)):
                        w = gbuf[b, e0 + t, pl.ds(h * 2 * L, 2 * L)]
                        pa, pb = plsc.unpack(
                            w, format=plsc.PackFormat.INTERLEAVED)
                        sbuf[sb, e0 + t, pl.ds(h * 2 * L, L)] = pa * v
                        sbuf[sb, e0 + t, pl.ds(h * 2 * L + L, L)] = pb * v
            pltpu.async_copy(sbuf.at[sb], acc_sh.at[rowb.at[u, b]],
                             ssem.at[sb], add=True)

            @pl.when(j + NBUF < NCHUNK)
            def _():
                pltpu.async_copy(src.at[colb.at[un, b]], gbuf.at[b],
                                 gsem.at[b])

            @pl.when(jnp.logical_and(b == 1, g < NGRP - 2))
            def _():
                _eb_issue(g + 2, uf)

            @pl.when(jnp.logical_and(b == NBUF - 1, g < NGRP - 2))
            def _():
                _eb_wait(g + 2, uf)
            return carry
        lax.fori_loop(0, NCHUNK, _chunk, 0)

        ul = (NGRP - 1) % EB
        pltpu.make_async_copy(sbuf.at[0], acc_sh.at[rowb.at[ul, 2]],
                              ssem.at[0]).wait()
        pltpu.make_async_copy(sbuf.at[1], acc_sh.at[rowb.at[ul, 3]],
                              ssem.at[1]).wait()
        plsc.subcore_barrier()

        nw = ROWS_PER_SUB // CH

        def _wsl(i):
            return pl.ds(s * ROWS_PER_SUB + i * CH, CH)

        def _wb_step(i, p):
            pltpu.make_async_copy(acc_sh.at[_wsl(i)], sbuf.at[p],
                                  wisem.at[p]).wait()

            @pl.when(i + 1 < nw)
            def _():
                pltpu.async_copy(acc_sh.at[_wsl(i + 1)], sbuf.at[1 - p],
                                 wisem.at[1 - p])
            if l < N_LAYERS - 1:
                pltpu.async_copy(zbuf, acc_sh.at[_wsl(i)], wsem)

            @pl.when(i >= 2)
            def _():
                pltpu.make_async_copy(
                    wbuf.at[p], xs_hbm.at[c, l, _wsl(i - 2)],
                    wosem.at[p]).wait()

            @plsc.parallel_loop(0, CH, step=L)
            def _packrow(e0):
                for t in range(L):
                    for h in range(D // (2 * L)):
                        pa = sbuf[p, e0 + t, pl.ds(h * 2 * L, L)]
                        pb = sbuf[p, e0 + t, pl.ds(h * 2 * L + L, L)]
                        wbuf[p, e0 + t, pl.ds(h * 2 * L, 2 * L)] = plsc.pack(
                            pa, pb, format=plsc.PackFormat.INTERLEAVED)
            pltpu.async_copy(wbuf.at[p], xs_hbm.at[c, l, _wsl(i)],
                             wosem.at[p])

        pltpu.async_copy(acc_sh.at[_wsl(0)], sbuf.at[0], wisem.at[0])

        def _wb2(i2, carry2):
            _wb_step(i2 * 2, 0)
            _wb_step(i2 * 2 + 1, 1)
            return carry2
        lax.fori_loop(0, nw // 2, _wb2, 0)
        for i in (nw - 2, nw - 1):
            p = i % 2
            pltpu.make_async_copy(wbuf.at[p], xs_hbm.at[c, l, _wsl(i)],
                                  wosem.at[p]).wait()
        if l < N_LAYERS - 1:
            for i in range(nw):
                pltpu.make_async_copy(zbuf, acc_sh.at[_wsl(i)], wsem).wait()
        plsc.subcore_barrier()


@functools.partial(
    pl.kernel,
    out_type=jax.ShapeDtypeStruct((BATCH,), jnp.float32),
    mesh=_mesh,
    scratch_types=[
        pltpu.VMEM((BPW,), jnp.int32),
        pltpu.VMEM((BPW,), jnp.int32),
        pltpu.VMEM((BPW,), jnp.int32),
        pltpu.VMEM((NUM_ITEMS,), jnp.int32),
        pltpu.VMEM((BPW, D), jnp.float32),
        pltpu.VMEM((BPW, D), jnp.float32),
        pltpu.VMEM((8, BPW, D), jnp.bfloat16),
        pltpu.VMEM((BPW,), jnp.float32),
        pltpu.SemaphoreType.DMA((8,)),
    ],
    compiler_params=_params,
)
def _readout(users_hbm, items_hbm, map_hbm, x0s_hbm, xs_hbm, gamma_hbm,
             uidx, iidx, midx, map_v, uacc, iacc, tmp8, gout, sem):
    c = lax.axis_index("c")
    s = lax.axis_index("s")
    wid = s * NC + c
    base = wid * BPW
    pltpu.sync_copy(users_hbm.at[pl.ds(base, BPW)], uidx)
    pltpu.sync_copy(items_hbm.at[pl.ds(base, BPW)], iidx)
    pltpu.sync_copy(map_hbm, map_v)

    for g in range(BPW // L):
        ivec = iidx[pl.ds(g * L, L)]
        m = plsc.load_gather(map_v, [ivec])
        midx[pl.ds(g * L, L)] = m + NUM_USERS
        iidx[pl.ds(g * L, L)] = ivec + NUM_USERS

    tables = [x0s_hbm.at[0], x0s_hbm.at[1]] + \
        [xs_hbm.at[cc, ll] for ll in range(N_LAYERS) for cc in range(NC)]

    def _acc_into(accbuf, gathered):
        for k, tbl in enumerate(gathered):
            pltpu.async_copy(tbl, tmp8.at[k], sem.at[k])
        for k, tbl in enumerate(gathered):
            pltpu.make_async_copy(tbl, tmp8.at[k], sem.at[k]).wait()

        @plsc.parallel_loop(0, BPW, step=L)
        def _zrow(e0):
            for t in range(L):
                for q in range(D // L):
                    accbuf[e0 + t, pl.ds(q * L, L)] = jnp.zeros(
                        (L,), jnp.float32)

        def _slab(k, carry):
            @plsc.parallel_loop(0, BPW, step=L)
            def _addrow(e0):
                for t in range(L):
                    for h in range(D // (2 * L)):
                        w = tmp8[k, e0 + t, pl.ds(h * 2 * L, 2 * L)]
                        pa, pb = plsc.unpack(
                            w, format=plsc.PackFormat.INTERLEAVED)
                        sla = pl.ds(h * 2 * L, L)
                        slb = pl.ds(h * 2 * L + L, L)
                        accbuf[e0 + t, sla] = accbuf[e0 + t, sla] + pa
                        accbuf[e0 + t, slb] = accbuf[e0 + t, slb] + pb
            return carry
        lax.fori_loop(0, 8, _slab, 0)

    _acc_into(uacc, [tb.at[uidx] for tb in tables])
    _acc_into(iacc, [tb.at[iidx if k % 2 == 0 else midx]
                     for k, tb in enumerate(tables)])

    lane = lax.broadcasted_iota(jnp.int32, (L,), 0)

    def _dot(g, carry):
        gvec = jnp.zeros((L,), jnp.float32)
        for t in range(L):
            e = g * L + t
            p = uacc[e, pl.ds(0, L)] * iacc[e, pl.ds(0, L)]
            for q in range(1, D // L):
                sl = pl.ds(q * L, L)
                p = p + uacc[e, sl] * iacc[e, sl]
            gvec = jnp.where(lane == t, jnp.sum(p) * (1.0 / 64.0), gvec)
        gout[pl.ds(g * L, L)] = gvec
        return carry
    lax.fori_loop(0, BPW // L, _dot, 0)
    pltpu.sync_copy(gout, gamma_hbm.at[pl.ds(base, BPW)])


def _pad_edges(a):
    return jnp.pad(a, (0, E_PAD - NNZ)).reshape(NS, NCHUNK, CH)


def kernel(users, items, user_emb_0, user_emb_1, item_emb_0, item_emb_1,
           edge_row_0, edge_col_0, edge_val_0,
           edge_row_1, edge_col_1, edge_val_1, map_list):
    x0s = jnp.stack([jnp.concatenate([user_emb_0, item_emb_0], axis=0),
                     jnp.concatenate([user_emb_1, item_emb_1], axis=0)])
    x0b = (x0s.reshape(NC, N, 2, 2, L).transpose(0, 1, 2, 4, 3)
           .reshape(NC, N, D).astype(jnp.bfloat16))
    colp = jnp.stack([_pad_edges(edge_col_0), _pad_edges(edge_col_1)])
    rowp = jnp.stack([_pad_edges(edge_row_0), _pad_edges(edge_row_1)])
    valp = jnp.stack([_pad_edges(edge_val_0), _pad_edges(edge_val_1)])
    xs = _propagate(x0b, colp, rowp, valp)
    return _readout(users, items, map_list, x0b, xs)
